# baseline (speedup 1.0000x reference)
import numpy as np
import jax
import jax.numpy as jnp
from jax import lax
from jax.experimental import pallas as pl
from jax.experimental.pallas import tpu as pltpu
from jax.experimental.pallas import tpu_sc as plsc

N_LEVELS = 16
N_FEATS = 2
TABLE_SIZE = 1 << 19
BASE_RES = 16.0
FINEST_RES = 512.0
D = 3
N_POINTS = 262144

NC = 2
NS = 16
NW = NC * NS
PTS_PER_W = N_POINTS // NW
C = 512
NCHUNK = PTS_PER_W // C
G = C // 16

N_COARSE = 4

_B = np.exp((np.log(FINEST_RES) - np.log(BASE_RES)) / (N_LEVELS - 1))
_RES = [int(np.floor(BASE_RES * _B**l)) for l in range(N_LEVELS)]
_GS = [np.float32(np.float32(2.0) / np.float32(r)) for r in _RES]
_INV_GS = [np.float32(np.float32(1.0) / g) for g in _GS]
_GS_POW2 = [float(g) == 2.0 ** round(np.log2(float(g))) for g in _GS]
_P1 = int(np.uint32(2654435761).astype(np.int32))
_P2 = int(np.uint32(805459861).astype(np.int32))
_MASK = (1 << 19) - 1

_SIDE = [r + 2 for r in _RES[:N_COARSE]]
_GSIZE = [((s * s * s + 15) // 16) * 16 for s in _SIDE]
_GOFF = [sum(_GSIZE[:i]) for i in range(N_COARSE)]
_GRID_WORDS = sum(_GSIZE)
_FILL_SEG = 4096


def _sc_body(xf_hbm, tf_hbm, out_hbm,
             xbuf, wbuf0, wbuf1, idxbuf0, idxbuf1, rowsbuf0, rowsbuf1,
             outbuf, gridbuf, sem0, sem1):
    wid = lax.axis_index("s") * NC + lax.axis_index("c")
    base0 = wid * PTS_PER_W
    wbufs = (wbuf0, wbuf1)
    idxbufs = (idxbuf0, idxbuf1)
    rowsbufs = (rowsbuf0, rowsbuf1)
    sems = (sem0, sem1)
    iota = lax.iota(jnp.int32, 16)

    for lvl in range(N_COARSE):
        side = _SIDE[lvl]
        n = side * side * side
        word_base = lvl * TABLE_SIZE
        segs = []
        off = 0
        while off < _GSIZE[lvl]:
            segs.append((off, min(_FILL_SEG, _GSIZE[lvl] - off)))
            off += segs[-1][1]
        for (soff, slen) in segs:
            @pl.loop(0, slen // 16)
            def _(i, _soff=soff, _side=side, _n=n, _wb=word_base):
                v = jnp.minimum(iota + (_soff + i * 16), jnp.int32(_n - 1))
                t = v // jnp.int32(_side)
                cz = v - t * jnp.int32(_side)
                a = t // jnp.int32(_side)
                cy = t - a * jnp.int32(_side)
                h = a ^ (cy * jnp.int32(_P1)) ^ (cz * jnp.int32(_P2))
                idxbuf0[pl.ds(i * 16, 16)] = (
                    (h & jnp.int32(_MASK)) + jnp.int32(_wb))
            pltpu.async_copy(
                tf_hbm.at[idxbuf0.at[pl.ds(0, slen)]],
                gridbuf.at[pl.ds(_GOFF[lvl] + soff, slen)], sem0).wait()

    def bl_and_w(lvl, s, store_w, wb=None):
        gs = _GS[lvl]
        inv_gs = _INV_GS[lvl]
        bls, ws = [], []
        for d in range(D):
            xd = xbuf[pl.ds(d * C + s, 16)]
            x1 = xd + jnp.float32(1.0)
            t = x1 * inv_gs if _GS_POW2[lvl] else x1 / gs
            bl = t.astype(jnp.int32)
            blf = bl.astype(jnp.float32)
            vmin = blf * gs + jnp.float32(-1.0)
            w = (xd - vmin) * inv_gs
            if store_w:
                wb[pl.ds(d * C + s, 16)] = w
            else:
                ws.append(w)
            bls.append(bl)
        return bls, ws

    def unpack_word(pw):
        bb = plsc.bitcast(pw, jnp.bfloat16)
        return plsc.unpack(bb, format=plsc.PackFormat.INTERLEAVED,
                           preferred_element_type=jnp.float32)

    def coarse_level(lvl):
        side = _SIDE[lvl]
        goff = _GOFF[lvl]

        @pl.loop(0, G)
        def _(i):
            s = i * 16
            (bl0, bl1, bl2), (w0, w1, w2) = bl_and_w(lvl, s, False)
            w01 = w0 * w1
            w02 = w0 * w2
            w12 = w1 * w2
            w012 = w01 * w2
            wcs = (None, w2, w1, w12, w0, w02, w01, w012)
            k000 = ((bl0 * jnp.int32(side) + bl1) * jnp.int32(side)
                    + bl2 + jnp.int32(goff))
            acc0 = acc1 = None
            for j in range(8):
                b2, b1, b0 = (j >> 2) & 1, (j >> 1) & 1, j & 1
                kj = k000 + jnp.int32(b2 * side * side + b1 * side + b0)
                f0, f1 = unpack_word(plsc.load_gather(gridbuf, [kj]))
                if j == 0:
                    acc0, acc1 = f0, f1
                else:
                    acc0 = acc0 + f0 * wcs[j]
                    acc1 = acc1 + f1 * wcs[j]
            outbuf[2 * lvl, pl.ds(s, 16)] = acc0
            outbuf[2 * lvl + 1, pl.ds(s, 16)] = acc1

    def compute_idx(lvl, wb, idxb):
        word_base = lvl * TABLE_SIZE

        @pl.loop(0, G)
        def _(i):
            s = i * 16
            (bl0, bl1, bl2), _ = bl_and_w(lvl, s, True, wb)
            m1 = bl1 * jnp.int32(_P1)
            m1b = m1 + jnp.int32(_P1)
            m2 = bl2 * jnp.int32(_P2)
            m2b = m2 + jnp.int32(_P2)
            bl0b = bl0 + jnp.int32(1)
            e = (bl0 ^ m1, bl0 ^ m1b, bl0b ^ m1, bl0b ^ m1b)
            for j in range(8):
                b2, b1, b0 = (j >> 2) & 1, (j >> 1) & 1, j & 1
                h = e[2 * b2 + b1] ^ (m2b if b0 else m2)
                idxb[pl.ds(j * C + s, 16)] = (
                    (h & jnp.int32(0xFFF)) + jnp.int32(word_base))

    def interp(lvl, wb, rowsb):
        @pl.loop(0, G)
        def _(i):
            s = i * 16
            w0 = wb[pl.ds(s, 16)]
            w1 = wb[pl.ds(C + s, 16)]
            w2 = wb[pl.ds(2 * C + s, 16)]
            w01 = w0 * w1
            w02 = w0 * w2
            w12 = w1 * w2
            w012 = w01 * w2
            wcs = (None, w2, w1, w12, w0, w02, w01, w012)
            acc0, acc1 = unpack_word(rowsb[pl.ds(s, 16)])
            for j in range(1, 8):
                f0, f1 = unpack_word(rowsb[pl.ds(j * C + s, 16)])
                acc0 = acc0 + f0 * wcs[j]
                acc1 = acc1 + f1 * wcs[j]
            outbuf[2 * lvl, pl.ds(s, 16)] = acc0
            outbuf[2 * lvl + 1, pl.ds(s, 16)] = acc1

    @pl.loop(0, NCHUNK)
    def _(ci):
        base = base0 + ci * C
        for d in range(D):
            pltpu.sync_copy(xf_hbm.at[pl.ds(d * N_POINTS + base, C)],
                            xbuf.at[pl.ds(d * C, C)])
        def start_gather(p):
            return [pltpu.async_copy(
                tf_hbm.at[idxbufs[p].at[pl.ds(q * 2 * C, 2 * C)]],
                rowsbufs[p].at[pl.ds(q * 2 * C, 2 * C)], sems[p])
                for q in range(4)]

        compute_idx(N_COARSE, wbufs[0], idxbufs[0])
        copies = {N_COARSE: start_gather(0)}
        for lvl in range(N_COARSE):
            coarse_level(lvl)
        for l in range(N_COARSE, N_LEVELS):
            p = (l + 1 - N_COARSE) % 2
            if l + 1 < N_LEVELS:
                compute_idx(l + 1, wbufs[p], idxbufs[p])
                copies[l + 1] = start_gather(p)
            for cp in copies[l]:
                cp.wait()
            q = (l - N_COARSE) % 2
            interp(l, wbufs[q], rowsbufs[q])
        pltpu.sync_copy(outbuf, out_hbm.at[:, pl.ds(base, C)])


def _build_kernel(interpret=False):
    mesh = plsc.VectorSubcoreMesh(core_axis_name="c", subcore_axis_name="s",
                                  num_cores=NC, num_subcores=NS)
    return pl.kernel(
        _sc_body,
        out_type=jax.ShapeDtypeStruct((2 * N_LEVELS, N_POINTS), jnp.float32),
        mesh=mesh,
        interpret=interpret,
        compiler_params=pltpu.CompilerParams(needs_layout_passes=False),
        scratch_types=[
            pltpu.VMEM((D * C,), jnp.float32),
            pltpu.VMEM((D * C,), jnp.float32),
            pltpu.VMEM((D * C,), jnp.float32),
            pltpu.VMEM((8 * C,), jnp.int32),
            pltpu.VMEM((8 * C,), jnp.int32),
            pltpu.VMEM((8 * C,), jnp.int32),
            pltpu.VMEM((8 * C,), jnp.int32),
            pltpu.VMEM((2 * N_LEVELS, C), jnp.float32),
            pltpu.VMEM((_GRID_WORDS,), jnp.int32),
            pltpu.SemaphoreType.DMA,
            pltpu.SemaphoreType.DMA,
        ],
    )


@jax.jit
def kernel(x, tables):
    xf = x.T.reshape(D * N_POINTS)
    tf = jax.lax.bitcast_convert_type(
        tables.astype(jnp.bfloat16), jnp.int32).reshape(N_LEVELS * TABLE_SIZE)
    out_t = _build_kernel()(xf, tf)
    return out_t.T


# speedup vs baseline: 1.3017x; 1.3017x over previous
import numpy as np
import jax
import jax.numpy as jnp
from jax import lax
from jax.experimental import pallas as pl
from jax.experimental.pallas import tpu as pltpu
from jax.experimental.pallas import tpu_sc as plsc

N_LEVELS = 16
N_FEATS = 2
TABLE_SIZE = 1 << 19
BASE_RES = 16.0
FINEST_RES = 512.0
D = 3
N_POINTS = 262144

NC = 2
NS = 16
NW = NC * NS
PTS_PER_W = N_POINTS // NW
C = 512
NCHUNK = PTS_PER_W // C
G = C // 16

N_COARSE = 4

_B = np.exp((np.log(FINEST_RES) - np.log(BASE_RES)) / (N_LEVELS - 1))
_RES = [int(np.floor(BASE_RES * _B**l)) for l in range(N_LEVELS)]
_GS = [np.float32(np.float32(2.0) / np.float32(r)) for r in _RES]
_INV_GS = [np.float32(np.float32(1.0) / g) for g in _GS]
_GS_POW2 = [float(g) == 2.0 ** round(np.log2(float(g))) for g in _GS]
_P1 = int(np.uint32(2654435761).astype(np.int32))
_P2 = int(np.uint32(805459861).astype(np.int32))
_MASK = (1 << 19) - 1

_SIDE = [r + 2 for r in _RES[:N_COARSE]]
_GSIZE = [((s * s * s + 15) // 16) * 16 for s in _SIDE]
_GOFF = [sum(_GSIZE[:i]) for i in range(N_COARSE)]
_GRID_WORDS = sum(_GSIZE)
_FILL_SEG = 4096


def _sc_body(xf_hbm, tf_hbm, out_hbm,
             xbuf, wbuf0, wbuf1, idxbuf0, idxbuf1, rowsbuf0, rowsbuf1,
             outbuf, gridbuf, sem0, sem1):
    wid = lax.axis_index("s") * NC + lax.axis_index("c")
    base0 = wid * PTS_PER_W
    wbufs = (wbuf0, wbuf1)
    idxbufs = (idxbuf0, idxbuf1)
    rowsbufs = (rowsbuf0, rowsbuf1)
    sems = (sem0, sem1)
    iota = lax.iota(jnp.int32, 16)

    for lvl in range(N_COARSE):
        side = _SIDE[lvl]
        n = side * side * side
        word_base = lvl * TABLE_SIZE
        segs = []
        off = 0
        while off < _GSIZE[lvl]:
            segs.append((off, min(_FILL_SEG, _GSIZE[lvl] - off)))
            off += segs[-1][1]
        for (soff, slen) in segs:
            @pl.loop(0, slen // 16)
            def _(i, _soff=soff, _side=side, _n=n, _wb=word_base):
                v = jnp.minimum(iota + (_soff + i * 16), jnp.int32(_n - 1))
                t = v // jnp.int32(_side)
                cz = v - t * jnp.int32(_side)
                a = t // jnp.int32(_side)
                cy = t - a * jnp.int32(_side)
                h = a ^ (cy * jnp.int32(_P1)) ^ (cz * jnp.int32(_P2))
                idxbuf0[pl.ds(i * 16, 16)] = (
                    (h & jnp.int32(_MASK)) + jnp.int32(_wb))
            pltpu.async_copy(
                tf_hbm.at[idxbuf0.at[pl.ds(0, slen)]],
                gridbuf.at[pl.ds(_GOFF[lvl] + soff, slen)], sem0).wait()

    def bl_and_w(lvl, s, store_w, wb=None):
        gs = _GS[lvl]
        inv_gs = _INV_GS[lvl]
        bls, ws = [], []
        for d in range(D):
            xd = xbuf[pl.ds(d * C + s, 16)]
            x1 = xd + jnp.float32(1.0)
            t = x1 * inv_gs if _GS_POW2[lvl] else x1 / gs
            bl = t.astype(jnp.int32)
            blf = bl.astype(jnp.float32)
            vmin = blf * gs + jnp.float32(-1.0)
            w = (xd - vmin) * inv_gs
            if store_w:
                wb[pl.ds(d * C + s, 16)] = w
            else:
                ws.append(w)
            bls.append(bl)
        return bls, ws

    def unpack_word(pw):
        bb = plsc.bitcast(pw, jnp.bfloat16)
        return plsc.unpack(bb, format=plsc.PackFormat.INTERLEAVED,
                           preferred_element_type=jnp.float32)

    def coarse_level(lvl):
        side = _SIDE[lvl]
        goff = _GOFF[lvl]

        @pl.loop(0, G)
        def _(i):
            s = i * 16
            (bl0, bl1, bl2), (w0, w1, w2) = bl_and_w(lvl, s, False)
            w01 = w0 * w1
            w02 = w0 * w2
            w12 = w1 * w2
            w012 = w01 * w2
            wcs = (None, w2, w1, w12, w0, w02, w01, w012)
            k000 = ((bl0 * jnp.int32(side) + bl1) * jnp.int32(side)
                    + bl2 + jnp.int32(goff))
            acc0 = acc1 = None
            for j in range(8):
                b2, b1, b0 = (j >> 2) & 1, (j >> 1) & 1, j & 1
                kj = k000 + jnp.int32(b2 * side * side + b1 * side + b0)
                f0, f1 = unpack_word(plsc.load_gather(gridbuf, [kj]))
                if j == 0:
                    acc0, acc1 = f0, f1
                else:
                    acc0 = acc0 + f0 * wcs[j]
                    acc1 = acc1 + f1 * wcs[j]
            outbuf[2 * lvl, pl.ds(s, 16)] = acc0
            outbuf[2 * lvl + 1, pl.ds(s, 16)] = acc1

    def compute_idx(lvl, wb, idxb):
        word_base = lvl * TABLE_SIZE

        @pl.loop(0, G)
        def _(i):
            s = i * 16
            (bl0, bl1, bl2), _ = bl_and_w(lvl, s, True, wb)
            m1 = bl1 * jnp.int32(_P1)
            m1b = m1 + jnp.int32(_P1)
            m2 = bl2 * jnp.int32(_P2)
            m2b = m2 + jnp.int32(_P2)
            bl0b = bl0 + jnp.int32(1)
            e = (bl0 ^ m1, bl0 ^ m1b, bl0b ^ m1, bl0b ^ m1b)
            for j in range(8):
                b2, b1, b0 = (j >> 2) & 1, (j >> 1) & 1, j & 1
                h = e[2 * b2 + b1] ^ (m2b if b0 else m2)
                idxb[pl.ds(j * C + s, 16)] = (
                    (h & jnp.int32(_MASK)) + jnp.int32(word_base))

    def interp(lvl, wb, rowsb):
        @pl.loop(0, G)
        def _(i):
            s = i * 16
            w0 = wb[pl.ds(s, 16)]
            w1 = wb[pl.ds(C + s, 16)]
            w2 = wb[pl.ds(2 * C + s, 16)]
            w01 = w0 * w1
            w02 = w0 * w2
            w12 = w1 * w2
            w012 = w01 * w2
            wcs = (None, w2, w1, w12, w0, w02, w01, w012)
            acc0, acc1 = unpack_word(rowsb[pl.ds(s, 16)])
            for j in range(1, 8):
                f0, f1 = unpack_word(rowsb[pl.ds(j * C + s, 16)])
                acc0 = acc0 + f0 * wcs[j]
                acc1 = acc1 + f1 * wcs[j]
            outbuf[2 * lvl, pl.ds(s, 16)] = acc0
            outbuf[2 * lvl + 1, pl.ds(s, 16)] = acc1

    @pl.loop(0, NCHUNK)
    def _(ci):
        base = base0 + ci * C
        for d in range(D):
            pltpu.sync_copy(xf_hbm.at[pl.ds(d * N_POINTS + base, C)],
                            xbuf.at[pl.ds(d * C, C)])
        compute_idx(N_COARSE, wbufs[0], idxbufs[0])
        copies = {N_COARSE: pltpu.async_copy(
            tf_hbm.at[idxbufs[0]], rowsbufs[0], sems[0])}
        for lvl in range(N_COARSE):
            coarse_level(lvl)
        for l in range(N_COARSE, N_LEVELS):
            p = (l + 1 - N_COARSE) % 2
            if l + 1 < N_LEVELS:
                compute_idx(l + 1, wbufs[p], idxbufs[p])
                copies[l + 1] = pltpu.async_copy(
                    tf_hbm.at[idxbufs[p]], rowsbufs[p], sems[p])
            copies[l].wait()
            q = (l - N_COARSE) % 2
            interp(l, wbufs[q], rowsbufs[q])
        pltpu.sync_copy(outbuf, out_hbm.at[:, pl.ds(base, C)])


def _build_kernel(interpret=False):
    mesh = plsc.VectorSubcoreMesh(core_axis_name="c", subcore_axis_name="s",
                                  num_cores=NC, num_subcores=NS)
    return pl.kernel(
        _sc_body,
        out_type=jax.ShapeDtypeStruct((2 * N_LEVELS, N_POINTS), jnp.float32),
        mesh=mesh,
        interpret=interpret,
        compiler_params=pltpu.CompilerParams(needs_layout_passes=False),
        scratch_types=[
            pltpu.VMEM((D * C,), jnp.float32),
            pltpu.VMEM((D * C,), jnp.float32),
            pltpu.VMEM((D * C,), jnp.float32),
            pltpu.VMEM((8 * C,), jnp.int32),
            pltpu.VMEM((8 * C,), jnp.int32),
            pltpu.VMEM((8 * C,), jnp.int32),
            pltpu.VMEM((8 * C,), jnp.int32),
            pltpu.VMEM((2 * N_LEVELS, C), jnp.float32),
            pltpu.VMEM((_GRID_WORDS,), jnp.int32),
            pltpu.SemaphoreType.DMA,
            pltpu.SemaphoreType.DMA,
        ],
    )


@jax.jit
def kernel(x, tables):
    xf = x.T.reshape(D * N_POINTS)
    tf = jax.lax.bitcast_convert_type(
        tables.astype(jnp.bfloat16), jnp.int32).reshape(N_LEVELS * TABLE_SIZE)
    out_t = _build_kernel()(xf, tf)
    return out_t.T
